# initial kernel scaffold (unmeasured)
import jax
import jax.numpy as jnp
from jax import lax
from jax.experimental import pallas as pl
from jax.experimental.pallas import tpu as pltpu

N_DEV = 16


def kernel(x, w_mat):
    m, k = x.shape
    k2, n = w_mat.shape
    assert k == k2
    m_chunk = m // N_DEV

    def body(x_ref, w_ref, out_ref, sendbuf, recvbuf, send_sems, recv_sems,
             credit_sem):
        my = lax.axis_index("i")
        left = lax.rem(my + N_DEV - 1, N_DEV)
        right = lax.rem(my + 1, N_DEV)

        barrier_sem = pltpu.get_barrier_semaphore()
        for nbr in (left, right):
            pl.semaphore_signal(
                barrier_sem, inc=1,
                device_id=(nbr,), device_id_type=pl.DeviceIdType.MESH,
            )
        pl.semaphore_wait(barrier_sem, 2)

        out_ref[:, :] = lax.dot_general(
            x_ref[:, :].astype(jnp.bfloat16),
            w_ref[:, :].astype(jnp.bfloat16),
            (((1,), (0,)), ((), ())),
            preferred_element_type=jnp.float32,
        )

        def chunk(ref, c):
            return ref[pl.ds(c * m_chunk, m_chunk), :]

        n_steps = 2 * (N_DEV - 1) + 2
        for t in range(n_steps):
            slot = t % 2
            send_c = lax.rem(my - t + 4 * N_DEV, N_DEV)
            recv_c = lax.rem(my - t - 1 + 4 * N_DEV, N_DEV)

            if t == N_DEV - 1:
                oc = lax.rem(my + 1, N_DEV)
                v = chunk(out_ref, oc)
                out_ref[pl.ds(oc * m_chunk, m_chunk), :] = v * jax.nn.sigmoid(v)

            if t >= 2:
                pl.semaphore_wait(credit_sem, 1)

            sendbuf[slot, :, :] = chunk(out_ref, send_c).astype(jnp.bfloat16)
            rdma = pltpu.make_async_remote_copy(
                src_ref=sendbuf.at[slot],
                dst_ref=recvbuf.at[slot],
                send_sem=send_sems.at[slot],
                recv_sem=recv_sems.at[slot],
                device_id=(right,),
                device_id_type=pl.DeviceIdType.MESH,
            )
            rdma.start()
            rdma.wait()

            got = recvbuf[slot, :, :].astype(jnp.float32)
            if t < N_DEV - 1:
                out_ref[pl.ds(recv_c * m_chunk, m_chunk), :] = (
                    chunk(out_ref, recv_c) + got
                )
            else:
                out_ref[pl.ds(recv_c * m_chunk, m_chunk), :] = got

            if t <= n_steps - 3:
                pl.semaphore_signal(
                    credit_sem, inc=1,
                    device_id=(left,), device_id_type=pl.DeviceIdType.MESH,
                )

    return pl.pallas_call(
        body,
        out_shape=jax.ShapeDtypeStruct((m, n), jnp.float32),
        in_specs=[
            pl.BlockSpec(memory_space=pltpu.VMEM),
            pl.BlockSpec(memory_space=pltpu.VMEM),
        ],
        out_specs=pl.BlockSpec(memory_space=pltpu.VMEM),
        scratch_shapes=[
            pltpu.VMEM((2, m // N_DEV, n), jnp.bfloat16),
            pltpu.VMEM((2, m // N_DEV, n), jnp.bfloat16),
            pltpu.SemaphoreType.DMA((2,)),
            pltpu.SemaphoreType.DMA((2,)),
            pltpu.SemaphoreType.REGULAR,
        ],
        compiler_params=pltpu.CompilerParams(collective_id=0),
    )(x, w_mat)


# baseline (device time: 445865 ns/iter reference)
import jax
import jax.numpy as jnp
from jax import lax
from jax.experimental import pallas as pl
from jax.experimental.pallas import tpu as pltpu

N_DEV = 16


def kernel(x, w_mat):
    m, k = x.shape
    k2, n = w_mat.shape
    assert k == k2
    m_chunk = m // N_DEV

    def body(x_ref, w_ref, out_ref, sendbuf, recvbuf, send_sems, recv_sems,
             credit_sem):
        my = lax.axis_index("i")
        left = lax.rem(my + N_DEV - 1, N_DEV)
        right = lax.rem(my + 1, N_DEV)

        barrier_sem = pltpu.get_barrier_semaphore()
        for nbr in (left, right):
            pl.semaphore_signal(
                barrier_sem, inc=1,
                device_id=(nbr,), device_id_type=pl.DeviceIdType.MESH,
            )
        pl.semaphore_wait(barrier_sem, 2)

        out_ref[:, :] = lax.dot_general(
            x_ref[:, :].astype(jnp.bfloat16),
            w_ref[:, :].astype(jnp.bfloat16),
            (((1,), (0,)), ((), ())),
            preferred_element_type=jnp.float32,
        )

        def chunk(ref, c):
            return ref[pl.ds(c * m_chunk, m_chunk), :]

        n_steps = 2 * (N_DEV - 1)
        for t in range(n_steps):
            slot = t % 2
            send_c = lax.rem(my - t + 4 * N_DEV, N_DEV)
            recv_c = lax.rem(my - t - 1 + 4 * N_DEV, N_DEV)

            if t == N_DEV - 1:
                oc = lax.rem(my + 1, N_DEV)
                v = chunk(out_ref, oc)
                out_ref[pl.ds(oc * m_chunk, m_chunk), :] = v * jax.nn.sigmoid(v)

            if t >= 2:
                pl.semaphore_wait(credit_sem, 1)

            sendbuf[slot, :, :] = chunk(out_ref, send_c).astype(jnp.bfloat16)
            rdma = pltpu.make_async_remote_copy(
                src_ref=sendbuf.at[slot],
                dst_ref=recvbuf.at[slot],
                send_sem=send_sems.at[slot],
                recv_sem=recv_sems.at[slot],
                device_id=(right,),
                device_id_type=pl.DeviceIdType.MESH,
            )
            rdma.start()
            rdma.wait()

            got = recvbuf[slot, :, :].astype(jnp.float32)
            if t < N_DEV - 1:
                out_ref[pl.ds(recv_c * m_chunk, m_chunk), :] = (
                    chunk(out_ref, recv_c) + got
                )
            else:
                out_ref[pl.ds(recv_c * m_chunk, m_chunk), :] = got

            if t <= n_steps - 3:
                pl.semaphore_signal(
                    credit_sem, inc=1,
                    device_id=(left,), device_id_type=pl.DeviceIdType.MESH,
                )

    return pl.pallas_call(
        body,
        out_shape=jax.ShapeDtypeStruct((m, n), jnp.float32),
        in_specs=[
            pl.BlockSpec(memory_space=pltpu.VMEM),
            pl.BlockSpec(memory_space=pltpu.VMEM),
        ],
        out_specs=pl.BlockSpec(memory_space=pltpu.VMEM),
        scratch_shapes=[
            pltpu.VMEM((2, m // N_DEV, n), jnp.bfloat16),
            pltpu.VMEM((2, m // N_DEV, n), jnp.bfloat16),
            pltpu.SemaphoreType.DMA((2,)),
            pltpu.SemaphoreType.DMA((2,)),
            pltpu.SemaphoreType.REGULAR,
        ],
        compiler_params=pltpu.CompilerParams(collective_id=0),
    )(x, w_mat)


# device time: 337595 ns/iter; 1.3207x vs baseline; 1.3207x over previous
import jax
import jax.numpy as jnp
from jax import lax
from jax.experimental import pallas as pl
from jax.experimental.pallas import tpu as pltpu

N_DEV = 16


def kernel(x, w_mat):
    m, k = x.shape
    k2, n = w_mat.shape
    assert k == k2
    mc = m // N_DEV
    mh = mc // 2

    def body(x_ref, w_ref, out_ref,
             xb, wb, sendA, recvA, sendB, recvB, ptlA, ptlB, stgA, stgB,
             sA_sems, rA_sems, sB_sems, rB_sems, stA_sems, stB_sems,
             credA, credB):
        my = lax.axis_index("i")
        left = lax.rem(my + N_DEV - 1, N_DEV)
        right = lax.rem(my + 1, N_DEV)

        barrier_sem = pltpu.get_barrier_semaphore()
        for nbr in (left, right):
            pl.semaphore_signal(
                barrier_sem, inc=1,
                device_id=(nbr,), device_id_type=pl.DeviceIdType.MESH,
            )
        pl.semaphore_wait(barrier_sem, 2)

        xb[:, :] = x_ref[:, :].astype(jnp.bfloat16)
        wb[:, :] = w_ref[:, :].astype(jnp.bfloat16)

        def partial(row_start):
            return lax.dot_general(
                xb[pl.ds(row_start, mh), :], wb[:, :],
                (((1,), (0,)), ((), ())),
                preferred_element_type=jnp.float32,
            )

        def hbm_store(stg, sems, slot, row_start):
            cp = pltpu.make_async_copy(
                stg.at[slot], out_ref.at[pl.ds(row_start, mh), :],
                sems.at[slot],
            )
            cp.start()
            cp.wait()

        def credit(sem, nbr):
            pl.semaphore_signal(
                sem, inc=1,
                device_id=(nbr,), device_id_type=pl.DeviceIdType.MESH,
            )

        ptlA[:, :] = partial(my * mc)
        ptlB[:, :] = partial(my * mc + mh)

        n_steps = 2 * (N_DEV - 1)
        for t in range(n_steps):
            slot = t % 2
            prev = (t - 1) % 2
            rcA = lax.rem(my - t - 1 + 4 * N_DEV, N_DEV)
            rcB = lax.rem(my + t + 1, N_DEV)

            if t < N_DEV - 1:
                sendA[slot, :, :] = ptlA[:, :].astype(jnp.bfloat16)
                sendB[slot, :, :] = ptlB[:, :].astype(jnp.bfloat16)
            elif t == N_DEV - 1:
                yA = ptlA[:, :] * jax.nn.sigmoid(ptlA[:, :])
                yB = ptlB[:, :] * jax.nn.sigmoid(ptlB[:, :])
                sendA[slot, :, :] = yA.astype(jnp.bfloat16)
                sendB[slot, :, :] = yB.astype(jnp.bfloat16)
                stgA[slot, :, :] = yA
                stgB[slot, :, :] = yB
                hbm_store(stgA, stA_sems, slot, lax.rem(my + 1, N_DEV) * mc)
                hbm_store(stgB, stB_sems, slot,
                          lax.rem(my + N_DEV - 1, N_DEV) * mc + mh)

            if t >= 2:
                pl.semaphore_wait(credA, 1)
                pl.semaphore_wait(credB, 1)

            srcA = sendA.at[slot] if t <= N_DEV - 1 else recvA.at[prev]
            srcB = sendB.at[slot] if t <= N_DEV - 1 else recvB.at[prev]
            rdmaA = pltpu.make_async_remote_copy(
                src_ref=srcA, dst_ref=recvA.at[slot],
                send_sem=sA_sems.at[slot], recv_sem=rA_sems.at[slot],
                device_id=(right,), device_id_type=pl.DeviceIdType.MESH,
            )
            rdmaB = pltpu.make_async_remote_copy(
                src_ref=srcB, dst_ref=recvB.at[slot],
                send_sem=sB_sems.at[slot], recv_sem=rB_sems.at[slot],
                device_id=(left,), device_id_type=pl.DeviceIdType.MESH,
            )
            rdmaA.start()
            rdmaB.start()

            if t < N_DEV - 1:
                npA = partial(rcA * mc)
                npB = partial(rcB * mc + mh)

            rdmaA.wait()
            rdmaB.wait()

            if t < N_DEV - 1:
                ptlA[:, :] = npA + recvA[slot, :, :].astype(jnp.float32)
                ptlB[:, :] = npB + recvB[slot, :, :].astype(jnp.float32)
                credit(credA, left)
                credit(credB, right)
            else:
                if N_DEV - 1 < t <= n_steps - 2:
                    credit(credA, left)
                    credit(credB, right)
                stgA[slot, :, :] = recvA[slot, :, :].astype(jnp.float32)
                stgB[slot, :, :] = recvB[slot, :, :].astype(jnp.float32)
                hbm_store(stgA, stA_sems, slot, rcA * mc)
                hbm_store(stgB, stB_sems, slot, rcB * mc + mh)

    return pl.pallas_call(
        body,
        out_shape=jax.ShapeDtypeStruct((m, n), jnp.float32),
        in_specs=[
            pl.BlockSpec(memory_space=pltpu.VMEM),
            pl.BlockSpec(memory_space=pltpu.VMEM),
        ],
        out_specs=pl.BlockSpec(memory_space=pl.ANY),
        scratch_shapes=[
            pltpu.VMEM((m, k), jnp.bfloat16),
            pltpu.VMEM((k, n), jnp.bfloat16),
            pltpu.VMEM((2, mc // 2, n), jnp.bfloat16),
            pltpu.VMEM((2, mc // 2, n), jnp.bfloat16),
            pltpu.VMEM((2, mc // 2, n), jnp.bfloat16),
            pltpu.VMEM((2, mc // 2, n), jnp.bfloat16),
            pltpu.VMEM((mc // 2, n), jnp.float32),
            pltpu.VMEM((mc // 2, n), jnp.float32),
            pltpu.VMEM((2, mc // 2, n), jnp.float32),
            pltpu.VMEM((2, mc // 2, n), jnp.float32),
            pltpu.SemaphoreType.DMA((2,)),
            pltpu.SemaphoreType.DMA((2,)),
            pltpu.SemaphoreType.DMA((2,)),
            pltpu.SemaphoreType.DMA((2,)),
            pltpu.SemaphoreType.DMA((2,)),
            pltpu.SemaphoreType.DMA((2,)),
            pltpu.SemaphoreType.REGULAR,
            pltpu.SemaphoreType.REGULAR,
        ],
        compiler_params=pltpu.CompilerParams(collective_id=0),
    )(x, w_mat)


# device time: 258325 ns/iter; 1.7260x vs baseline; 1.3069x over previous
import jax
import jax.numpy as jnp
from jax import lax
from jax.experimental import pallas as pl
from jax.experimental.pallas import tpu as pltpu

N_DEV = 16
SUBS = 2
LANES = 2 * SUBS
N_STEPS = 2 * (N_DEV - 1)


def kernel(x, w_mat):
    m, k = x.shape
    k2, n = w_mat.shape
    assert k == k2
    mc = m // N_DEV
    mh = mc // 2
    ms = mh // SUBS

    def body(x_ref, w_ref, out_ref,
             xb, wb, sendbuf, recvbuf, stg,
             send_sems, recv_sems, st_sems, cred_sems):
        my = lax.axis_index("i")
        left = lax.rem(my + N_DEV - 1, N_DEV)
        right = lax.rem(my + 1, N_DEV)

        barrier_sem = pltpu.get_barrier_semaphore()
        for nbr in (left, right):
            pl.semaphore_signal(
                barrier_sem, inc=1,
                device_id=(nbr,), device_id_type=pl.DeviceIdType.MESH,
            )
        pl.semaphore_wait(barrier_sem, 2)

        xb[:, :] = x_ref[:, :].astype(jnp.bfloat16)
        wb[:, :] = w_ref[:, :].astype(jnp.bfloat16)

        def lane_dir(ln):
            return ln // SUBS

        def lane_off(ln):
            return lane_dir(ln) * mh + (ln % SUBS) * ms

        def send_chunk(ln, t):
            if lane_dir(ln) == 0:
                return lax.rem(my - t + 4 * N_DEV, N_DEV)
            return lax.rem(my + t, N_DEV)

        def recv_chunk(ln, t):
            if lane_dir(ln) == 0:
                return lax.rem(my - t - 1 + 4 * N_DEV, N_DEV)
            return lax.rem(my + t + 1, N_DEV)

        def partial(ln, c):
            return lax.dot_general(
                xb[pl.ds(c * mc + lane_off(ln), ms), :], wb[:, :],
                (((1,), (0,)), ((), ())),
                preferred_element_type=jnp.float32,
            )

        def launch(ln, t):
            slot = t % 2
            if t >= 2:
                pl.semaphore_wait(cred_sems.at[ln], 1)
            if t <= N_DEV - 1:
                src = sendbuf.at[ln, slot]
            else:
                src = recvbuf.at[ln, (t - 1) % 2]
            tgt = right if lane_dir(ln) == 0 else left
            rdma = pltpu.make_async_remote_copy(
                src_ref=src, dst_ref=recvbuf.at[ln, slot],
                send_sem=send_sems.at[ln, slot],
                recv_sem=recv_sems.at[ln, slot],
                device_id=(tgt,), device_id_type=pl.DeviceIdType.MESH,
            )
            rdma.start()
            return rdma

        def credit(ln):
            nbr = left if lane_dir(ln) == 0 else right
            pl.semaphore_signal(
                cred_sems.at[ln], inc=1,
                device_id=(nbr,), device_id_type=pl.DeviceIdType.MESH,
            )

        pending_st = {}

        def hbm_store(ln, slot, row_start, val):
            key = (ln, slot)
            if key in pending_st:
                pending_st.pop(key).wait()
            stg[ln, slot, :, :] = val
            cp = pltpu.make_async_copy(
                stg.at[ln, slot], out_ref.at[pl.ds(row_start, ms)],
                st_sems.at[ln, slot],
            )
            cp.start()
            pending_st[key] = cp

        for ln in range(LANES):
            sendbuf[ln, 0, :, :] = partial(ln, my).astype(jnp.bfloat16)
        rdmas = [launch(ln, 0) for ln in range(LANES)]
        nxt = [partial(ln, recv_chunk(ln, 0)) for ln in range(LANES)]

        for t in range(N_STEPS):
            slot = t % 2
            for ln in range(LANES):
                rdmas[ln].wait()
                rc = recv_chunk(ln, t)
                if t < N_DEV - 1:
                    got = recvbuf[ln, slot, :, :].astype(jnp.float32)
                    val = nxt[ln] + got
                    if t < N_DEV - 2:
                        sendbuf[ln, 1 - slot, :, :] = val.astype(jnp.bfloat16)
                    else:
                        y = val * jax.nn.sigmoid(val)
                        sendbuf[ln, 1 - slot, :, :] = y.astype(jnp.bfloat16)
                        hbm_store(ln, 0, rc * mc + lane_off(ln), y)
                    credit(ln)
                else:
                    if N_DEV - 1 < t:
                        if t <= N_STEPS - 2:
                            credit(ln)
                    hbm_store(ln, slot, rc * mc + lane_off(ln),
                              recvbuf[ln, slot, :, :].astype(jnp.float32))
                if t < N_STEPS - 1:
                    rdmas[ln] = launch(ln, t + 1)
            if t + 1 < N_DEV - 1:
                nxt = [partial(ln, recv_chunk(ln, t + 1))
                       for ln in range(LANES)]

        for cp in pending_st.values():
            cp.wait()

    return pl.pallas_call(
        body,
        out_shape=jax.ShapeDtypeStruct((m, n), jnp.float32),
        in_specs=[
            pl.BlockSpec(memory_space=pltpu.VMEM),
            pl.BlockSpec(memory_space=pltpu.VMEM),
        ],
        out_specs=pl.BlockSpec(memory_space=pl.ANY),
        scratch_shapes=[
            pltpu.VMEM((m, k), jnp.bfloat16),
            pltpu.VMEM((k, n), jnp.bfloat16),
            pltpu.VMEM((LANES, 2, mc // 2 // SUBS, n), jnp.bfloat16),
            pltpu.VMEM((LANES, 2, mc // 2 // SUBS, n), jnp.bfloat16),
            pltpu.VMEM((LANES, 2, mc // 2 // SUBS, n), jnp.float32),
            pltpu.SemaphoreType.DMA((LANES, 2)),
            pltpu.SemaphoreType.DMA((LANES, 2)),
            pltpu.SemaphoreType.DMA((LANES, 2)),
            pltpu.SemaphoreType.REGULAR((LANES,)),
        ],
        compiler_params=pltpu.CompilerParams(collective_id=0),
    )(x, w_mat)
